# Initial kernel scaffold; baseline (speedup 1.0000x reference)
import jax
import jax.numpy as jnp
from jax.experimental import pallas as pl


def kernel(query, key, value, proj_dir):
    raise NotImplementedError("write your pallas kernel here")


# speedup vs baseline: 1.8878x; 1.8878x over previous
import math

import jax
import jax.numpy as jnp
from jax.experimental import pallas as pl
from jax.experimental.pallas import tpu as pltpu

NUM_PROJS = 7
BLK = 256
SAMPLE = 256
N = 4096
D = 64
NBUCKETS = 128
CHUNK = 256
NCHUNKS = N // CHUNK


def _rank_kernel(x_ref, p_ref, pos_ref, oh_ref, posf_ref):
    lane = jax.lax.broadcasted_iota(jnp.int32, (CHUNK, NBUCKETS), 1)
    encf = jnp.where(lane < NUM_PROJS, jnp.exp2(lane.astype(jnp.float32)), 0.0)

    def hash_body(c, t_carry):
        xc = x_ref[0, pl.ds(c * CHUNK, CHUNK), :]
        s = jnp.dot(xc, p_ref[...], preferred_element_type=jnp.float32)
        binf = jnp.sum(jnp.where(s > 0.0, encf, 0.0), axis=1, keepdims=True)
        bin_i = binf.astype(jnp.int32)
        hsh = bin_i ^ (bin_i >> 1)
        oh = (hsh == lane).astype(jnp.float32)
        oh_ref[pl.ds(c * CHUNK, CHUNK), :] = oh
        return t_carry + jnp.sum(oh, axis=0, keepdims=True)

    total = jax.lax.fori_loop(0, NCHUNKS, hash_body,
                              jnp.zeros((1, NBUCKETS), jnp.float32))
    iu = jax.lax.broadcasted_iota(jnp.int32, (NBUCKETS, NBUCKETS), 0)
    iw = jax.lax.broadcasted_iota(jnp.int32, (NBUCKETS, NBUCKETS), 1)
    cex = jnp.dot(total, (iu < iw).astype(jnp.float32),
                  preferred_element_type=jnp.float32)

    r0 = jax.lax.broadcasted_iota(jnp.int32, (CHUNK, CHUNK), 0)
    c0 = jax.lax.broadcasted_iota(jnp.int32, (CHUNK, CHUNK), 1)
    ltri = (c0 < r0).astype(jnp.float32)

    def rank_body(c, r_carry):
        oh = oh_ref[pl.ds(c * CHUNK, CHUNK), :]
        within = jnp.dot(ltri, oh, preferred_element_type=jnp.float32)
        tot = within + cex + r_carry
        posc = jnp.sum(tot * oh, axis=1, keepdims=True)
        posf_ref[pl.ds(c * CHUNK, CHUNK), :] = posc
        return r_carry + jnp.sum(oh, axis=0, keepdims=True)

    jax.lax.fori_loop(0, NCHUNKS, rank_body, jnp.zeros((1, NBUCKETS), jnp.float32))
    pos_ref[0] = posf_ref[...].astype(jnp.int32).reshape(N // 128, 128)


def _ranks(x, proj):
    bh = x.shape[0]
    out = pl.pallas_call(
        _rank_kernel,
        grid=(bh,),
        in_specs=[
            pl.BlockSpec((1, N, D), lambda r: (r, 0, 0)),
            pl.BlockSpec((D, NBUCKETS), lambda r: (0, 0)),
        ],
        out_specs=pl.BlockSpec((1, N // 128, 128), lambda r: (r, 0, 0)),
        out_shape=jax.ShapeDtypeStruct((bh, N // 128, 128), jnp.int32),
        scratch_shapes=[
            pltpu.VMEM((N, NBUCKETS), jnp.float32),
            pltpu.VMEM((N, 1), jnp.float32),
        ],
    )(x, proj)
    return out.reshape(bh, N)


def _attn_kernel(q_ref, k_ref, v_ref, ks_ref, vs_ref, o_ref):
    scale = D ** (-0.5)
    w = float(N) / float(SAMPLE)
    q = q_ref[0]
    qk1 = jax.lax.dot_general(q, k_ref[0], (((1,), (1,)), ((), ())),
                              preferred_element_type=jnp.float32) * scale
    m1 = jnp.max(qk1, axis=1, keepdims=True)
    p1 = jnp.exp(qk1 - m1)
    s1 = jnp.sum(p1, axis=1, keepdims=True)
    u1 = jnp.dot(p1, v_ref[0], preferred_element_type=jnp.float32)

    qk2 = jax.lax.dot_general(q, ks_ref[0], (((1,), (1,)), ((), ())),
                              preferred_element_type=jnp.float32) * scale
    m2 = jnp.max(qk2, axis=1, keepdims=True)
    p2 = jnp.exp(qk2 - m2)
    s2 = jnp.sum(p2, axis=1, keepdims=True)
    u2 = jnp.dot(p2, vs_ref[0], preferred_element_type=jnp.float32)

    mm = jnp.maximum(m1, m2)
    a1 = jnp.exp(m1 - mm)
    a2 = w * jnp.exp(m2 - mm)
    o_ref[0] = (a1 * u1 + a2 * u2) / (a1 * s1 + a2 * s2)


def _attention(qs, ks, vs, ksub, vsub):
    bh = qs.shape[0]
    ng = N // BLK
    return pl.pallas_call(
        _attn_kernel,
        grid=(bh, ng),
        in_specs=[
            pl.BlockSpec((1, BLK, D), lambda r, g: (r, g, 0)),
            pl.BlockSpec((1, BLK, D), lambda r, g: (r, g, 0)),
            pl.BlockSpec((1, BLK, D), lambda r, g: (r, g, 0)),
            pl.BlockSpec((1, SAMPLE, D), lambda r, g: (r, 0, 0)),
            pl.BlockSpec((1, SAMPLE, D), lambda r, g: (r, 0, 0)),
        ],
        out_specs=pl.BlockSpec((1, BLK, D), lambda r, g: (r, g, 0)),
        out_shape=jax.ShapeDtypeStruct((bh, N, D), jnp.float32),
    )(qs, ks, vs, ksub, vsub)


def kernel(query, key, value, proj_dir):
    b, h, n, d = query.shape
    bh = b * h
    proj = jnp.zeros((D, NBUCKETS), jnp.float32).at[:, :NUM_PROJS].set(
        proj_dir.reshape(D, NUM_PROJS))

    q = query.reshape(bh, n, d)
    k = key.reshape(bh, n, d)
    v = value.reshape(bh, n, d)

    pos_q = _ranks(q, proj)
    pos_k = _ranks(k, proj)

    row_off = jnp.arange(bh, dtype=jnp.int32)[:, None] * n
    pq_flat = (pos_q + row_off).reshape(-1)
    pk_flat = (pos_k + row_off).reshape(-1)
    qs = jnp.zeros((bh * n, d), jnp.float32).at[pq_flat].set(
        q.reshape(-1, d)).reshape(bh, n, d)
    ks = jnp.zeros((bh * n, d), jnp.float32).at[pk_flat].set(
        k.reshape(-1, d)).reshape(bh, n, d)
    vs = jnp.zeros((bh * n, d), jnp.float32).at[pk_flat].set(
        v.reshape(-1, d)).reshape(bh, n, d)

    sampled = jax.random.randint(jax.random.key(42), (b, h, SAMPLE), 0, n)
    sampled = sampled.reshape(bh, SAMPLE)
    ksub = jnp.take_along_axis(ks, sampled[..., None], axis=1)
    vsub = jnp.take_along_axis(vs, sampled[..., None], axis=1)

    outs = _attention(qs, ks, vs, ksub, vsub)

    out = jnp.take_along_axis(outs, pos_q[..., None], axis=1)
    return out.reshape(b, h, n, d)


# speedup vs baseline: 5.1189x; 2.7116x over previous
import functools
import math

import jax
import jax.numpy as jnp
from jax import lax
from jax.experimental import pallas as pl
from jax.experimental.pallas import tpu as pltpu
from jax.experimental.pallas import tpu_sc as plsc

NUM_PROJS = 7
BLK = 256
SAMPLE = 256
N = 4096
D = 64
NBUCKETS = 128
CHUNK = 256
NCHUNKS = N // CHUNK


def _rank_kernel(x_ref, p_ref, pos_ref, oh_ref, posf_ref):
    lane = jax.lax.broadcasted_iota(jnp.int32, (CHUNK, NBUCKETS), 1)
    encf = jnp.where(lane < NUM_PROJS, jnp.exp2(lane.astype(jnp.float32)), 0.0)

    def hash_body(c, t_carry):
        xc = x_ref[0, pl.ds(c * CHUNK, CHUNK), :]
        s = jnp.dot(xc, p_ref[...], preferred_element_type=jnp.float32)
        binf = jnp.sum(jnp.where(s > 0.0, encf, 0.0), axis=1, keepdims=True)
        bin_i = binf.astype(jnp.int32)
        hsh = bin_i ^ (bin_i >> 1)
        oh = (hsh == lane).astype(jnp.float32)
        oh_ref[pl.ds(c * CHUNK, CHUNK), :] = oh
        return t_carry + jnp.sum(oh, axis=0, keepdims=True)

    total = jax.lax.fori_loop(0, NCHUNKS, hash_body,
                              jnp.zeros((1, NBUCKETS), jnp.float32))
    iu = jax.lax.broadcasted_iota(jnp.int32, (NBUCKETS, NBUCKETS), 0)
    iw = jax.lax.broadcasted_iota(jnp.int32, (NBUCKETS, NBUCKETS), 1)
    cex = jnp.dot(total, (iu < iw).astype(jnp.float32),
                  preferred_element_type=jnp.float32)

    r0 = jax.lax.broadcasted_iota(jnp.int32, (CHUNK, CHUNK), 0)
    c0 = jax.lax.broadcasted_iota(jnp.int32, (CHUNK, CHUNK), 1)
    ltri = (c0 < r0).astype(jnp.float32)

    def rank_body(c, r_carry):
        oh = oh_ref[pl.ds(c * CHUNK, CHUNK), :]
        within = jnp.dot(ltri, oh, preferred_element_type=jnp.float32)
        tot = within + cex + r_carry
        posc = jnp.sum(tot * oh, axis=1, keepdims=True)
        posf_ref[pl.ds(c * CHUNK, CHUNK), :] = posc
        return r_carry + jnp.sum(oh, axis=0, keepdims=True)

    jax.lax.fori_loop(0, NCHUNKS, rank_body, jnp.zeros((1, NBUCKETS), jnp.float32))
    pos_ref[0] = posf_ref[...].astype(jnp.int32).reshape(N // 128, 128)


def _ranks(x, proj):
    bh = x.shape[0]
    out = pl.pallas_call(
        _rank_kernel,
        grid=(bh,),
        in_specs=[
            pl.BlockSpec((1, N, D), lambda r: (r, 0, 0)),
            pl.BlockSpec((D, NBUCKETS), lambda r: (0, 0)),
        ],
        out_specs=pl.BlockSpec((1, N // 128, 128), lambda r: (r, 0, 0)),
        out_shape=jax.ShapeDtypeStruct((bh, N // 128, 128), jnp.int32),
        scratch_shapes=[
            pltpu.VMEM((N, NBUCKETS), jnp.float32),
            pltpu.VMEM((N, 1), jnp.float32),
        ],
    )(x, proj)
    return out.reshape(bh, N)


def _attn_kernel(q_ref, k_ref, v_ref, ks_ref, vs_ref, o_ref):
    scale = D ** (-0.5)
    w = float(N) / float(SAMPLE)
    q = q_ref[0]
    qk1 = jax.lax.dot_general(q, k_ref[0], (((1,), (1,)), ((), ())),
                              preferred_element_type=jnp.float32) * scale
    m1 = jnp.max(qk1, axis=1, keepdims=True)
    p1 = jnp.exp(qk1 - m1)
    s1 = jnp.sum(p1, axis=1, keepdims=True)
    u1 = jnp.dot(p1, v_ref[0], preferred_element_type=jnp.float32)

    qk2 = jax.lax.dot_general(q, ks_ref[0], (((1,), (1,)), ((), ())),
                              preferred_element_type=jnp.float32) * scale
    m2 = jnp.max(qk2, axis=1, keepdims=True)
    p2 = jnp.exp(qk2 - m2)
    s2 = jnp.sum(p2, axis=1, keepdims=True)
    u2 = jnp.dot(p2, vs_ref[0], preferred_element_type=jnp.float32)

    mm = jnp.maximum(m1, m2)
    a1 = jnp.exp(m1 - mm)
    a2 = w * jnp.exp(m2 - mm)
    o_ref[0] = (a1 * u1 + a2 * u2) / (a1 * s1 + a2 * s2)


def _attention(qs, ks, vs, ksub, vsub):
    bh = qs.shape[0]
    ng = N // BLK
    return pl.pallas_call(
        _attn_kernel,
        grid=(bh, ng),
        in_specs=[
            pl.BlockSpec((1, BLK, D), lambda r, g: (r, g, 0)),
            pl.BlockSpec((1, BLK, D), lambda r, g: (r, g, 0)),
            pl.BlockSpec((1, BLK, D), lambda r, g: (r, g, 0)),
            pl.BlockSpec((1, SAMPLE, D), lambda r, g: (r, 0, 0)),
            pl.BlockSpec((1, SAMPLE, D), lambda r, g: (r, 0, 0)),
        ],
        out_specs=pl.BlockSpec((1, BLK, D), lambda r, g: (r, g, 0)),
        out_shape=jax.ShapeDtypeStruct((bh, N, D), jnp.float32),
    )(qs, ks, vs, ksub, vsub)


BH = 32
NCH = N // 128


def _sc_permute_kernel(q_hbm, k_hbm, v_hbm, pq_hbm, pk_hbm, pkl_hbm, sp_hbm,
                       qs_hbm, ks_hbm, vs_hbm, ksub_hbm, vsub_hbm,
                       posq_v, posk_v, pokl_v, samp_v, sidx_v, kidx_v,
                       buf_v, sub_v, sem, sem2):
    wid = lax.axis_index("c") * 16 + lax.axis_index("s")
    base = wid * N
    pltpu.sync_copy(pq_hbm.at[wid], posq_v)
    pltpu.sync_copy(pk_hbm.at[wid], posk_v)
    pltpu.sync_copy(pkl_hbm.at[pl.ds(base, N)], pokl_v)
    pltpu.sync_copy(sp_hbm.at[pl.ds(wid * SAMPLE, SAMPLE)], samp_v)

    def scatter_one(src, dst, posv):
        def body(c):
            pltpu.sync_copy(src.at[pl.ds(base + c * 128, 128)], buf_v)
            pltpu.async_copy(buf_v, dst.at[posv.at[c]], sem).wait()
        return body

    for src, dst, posv in ((q_hbm, qs_hbm, posq_v),
                           (k_hbm, ks_hbm, posk_v),
                           (v_hbm, vs_hbm, posk_v)):
        pl.loop(0, NCH)(scatter_one(src, dst, posv))

    @pl.loop(0, N // 16)
    def inv_body(j):
        p16 = pokl_v[pl.ds(j * 16, 16)]
        plsc.store_scatter(sidx_v, [p16], j * 16 + lax.iota(jnp.int32, 16))

    @pl.loop(0, SAMPLE // 16)
    def kidx_body(t):
        s16 = samp_v[pl.ds(t * 16, 16)]
        g16 = plsc.load_gather(sidx_v, [s16])
        kidx_v[t // 8, pl.ds((t % 8) * 16, 16)] = g16 + base

    for src, dst in ((k_hbm, ksub_hbm), (v_hbm, vsub_hbm)):
        @pl.loop(0, 2)
        def sub_body(hh, src=src, dst=dst):
            pltpu.async_copy(src.at[kidx_v.at[hh]], sub_v, sem2).wait()
            pltpu.sync_copy(sub_v, dst.at[pl.ds(wid * SAMPLE + hh * 128, 128)])


def _sc_permute(q, k, v, pos_qg, pos_kg, pos_kl, samp):
    f32 = jnp.float32
    run = pl.kernel(
        _sc_permute_kernel,
        mesh=plsc.VectorSubcoreMesh(core_axis_name="c", subcore_axis_name="s"),
        compiler_params=pltpu.CompilerParams(needs_layout_passes=False,
                                             use_tc_tiling_on_sc=False),
        out_type=(
            jax.ShapeDtypeStruct((BH * N, D), f32),
            jax.ShapeDtypeStruct((BH * N, D), f32),
            jax.ShapeDtypeStruct((BH * N, D), f32),
            jax.ShapeDtypeStruct((BH * SAMPLE, D), f32),
            jax.ShapeDtypeStruct((BH * SAMPLE, D), f32),
        ),
        scratch_types=[
            pltpu.VMEM((NCH, 128), jnp.int32),
            pltpu.VMEM((NCH, 128), jnp.int32),
            pltpu.VMEM((N,), jnp.int32),
            pltpu.VMEM((SAMPLE,), jnp.int32),
            pltpu.VMEM((N,), jnp.int32),
            pltpu.VMEM((2, 128), jnp.int32),
            pltpu.VMEM((128, D), f32),
            pltpu.VMEM((128, D), f32),
            pltpu.SemaphoreType.DMA,
            pltpu.SemaphoreType.DMA,
        ],
    )
    return run(q, k, v, pos_qg, pos_kg, pos_kl, samp)


def _sc_unsort_kernel(a_hbm, pq_hbm, out_hbm, posq_v, buf_v, sem):
    wid = lax.axis_index("c") * 16 + lax.axis_index("s")
    base = wid * N
    pltpu.sync_copy(pq_hbm.at[wid], posq_v)

    @pl.loop(0, NCH)
    def body(c):
        pltpu.async_copy(a_hbm.at[posq_v.at[c]], buf_v, sem).wait()
        pltpu.sync_copy(buf_v, out_hbm.at[pl.ds(base + c * 128, 128)])


def _sc_unsort(attns, pos_qg):
    run = pl.kernel(
        _sc_unsort_kernel,
        mesh=plsc.VectorSubcoreMesh(core_axis_name="c", subcore_axis_name="s"),
        compiler_params=pltpu.CompilerParams(needs_layout_passes=False,
                                             use_tc_tiling_on_sc=False),
        out_type=jax.ShapeDtypeStruct((BH * N, D), jnp.float32),
        scratch_types=[
            pltpu.VMEM((NCH, 128), jnp.int32),
            pltpu.VMEM((128, D), jnp.float32),
            pltpu.SemaphoreType.DMA,
        ],
    )
    return run(attns, pos_qg)


def kernel(query, key, value, proj_dir):
    b, h, n, d = query.shape
    bh = b * h
    proj = jnp.zeros((D, NBUCKETS), jnp.float32).at[:, :NUM_PROJS].set(
        proj_dir.reshape(D, NUM_PROJS))

    q = query.reshape(bh, n, d)
    k = key.reshape(bh, n, d)
    v = value.reshape(bh, n, d)

    pos_q = _ranks(q, proj)
    pos_k = _ranks(k, proj)

    row_off = jnp.arange(bh, dtype=jnp.int32)[:, None] * n
    pos_qg = (pos_q + row_off).reshape(bh, NCH, 128)
    pos_kg = (pos_k + row_off).reshape(bh, NCH, 128)
    pos_kl = pos_k.reshape(-1)

    sampled = jax.random.randint(jax.random.key(42), (b, h, SAMPLE), 0, n)
    sampled = sampled.reshape(-1).astype(jnp.int32)

    qs, ks, vs, ksub, vsub = _sc_permute(
        q.reshape(-1, d), k.reshape(-1, d), v.reshape(-1, d),
        pos_qg, pos_kg, pos_kl, sampled)

    outs = _attention(qs.reshape(bh, n, d), ks.reshape(bh, n, d),
                      vs.reshape(bh, n, d), ksub.reshape(bh, SAMPLE, d),
                      vsub.reshape(bh, SAMPLE, d))

    out = _sc_unsort(outs.reshape(-1, d), pos_qg)
    return out.reshape(b, h, n, d)
